# baseline (speedup 1.0000x reference)
import jax
import jax.numpy as jnp
from jax.experimental import pallas as pl

_B, _D, _K = 4096, 64, 32
_N = _B * _D
_R = 1024
_G = _N // _R
_MIN_BIN = 0.001
_MIN_DER = 0.001


def _softplus(v):
    return jnp.maximum(v, 0.0) + jnp.log1p(jnp.exp(-jnp.abs(v)))


def _body(x_ref, cx_ref, cy_ref, uw_ref, uh_ref, ud_ref, out_ref, lad_ref):
    K = _K
    x = x_ref[0, 0, :]
    cx = cx_ref[0, 0, :]
    cy = cy_ref[0, 0, :]
    z = x - cx

    spw = _MIN_BIN + _softplus(uw_ref[...])
    sph = _MIN_BIN + _softplus(uh_ref[...])

    kio = jax.lax.broadcasted_iota(jnp.int32, (K, K + 1), 0)
    jio = jax.lax.broadcasted_iota(jnp.int32, (K, K + 1), 1)
    M = jnp.where(kio < jio, 0.5, -0.5).astype(jnp.float32)
    cw = jax.lax.dot_general(spw, M, (((1,), (0,)), ((), ())),
                             preferred_element_type=jnp.float32,
                             precision=jax.lax.Precision.HIGHEST)
    ch = jax.lax.dot_general(sph, M, (((1,), (0,)), ((), ())),
                             preferred_element_type=jnp.float32,
                             precision=jax.lax.Precision.HIGHEST)

    e0 = cw[:, 0]
    eK = cw[:, K]
    ch0 = ch[:, 0]
    chK = ch[:, K]

    lm = z < e0
    rm = z >= eK
    im = jnp.logical_not(jnp.logical_or(lm, rm))
    zst = jnp.where(im, z, 0.0)

    ind = (zst[:, None] >= cw).astype(jnp.float32)
    idx = jnp.sum(ind, axis=-1).astype(jnp.int32) - 1

    lane33 = jax.lax.broadcasted_iota(jnp.int32, (_R, K + 1), 1)
    lane32 = jax.lax.broadcasted_iota(jnp.int32, (_R, K), 1)
    idx2 = idx[:, None]
    oh_lo33 = lane33 == idx2
    oh_hi33 = lane33 == idx2 + 1
    oh32 = lane32 == idx2

    ud = ud_ref[...]
    cw_lo = jnp.sum(jnp.where(oh_lo33, cw, 0.0), axis=-1)
    ch_lo = jnp.sum(jnp.where(oh_lo33, ch, 0.0), axis=-1)
    w_b = jnp.sum(jnp.where(oh32, spw, 0.0), axis=-1)
    h_b = jnp.sum(jnp.where(oh32, sph, 0.0), axis=-1)
    d_lo_raw = jnp.sum(jnp.where(oh_lo33, ud, 0.0), axis=-1)
    d_hi_raw = jnp.sum(jnp.where(oh_hi33, ud, 0.0), axis=-1)
    d0_raw = ud[:, 0]
    dK_raw = ud[:, K]

    d_lo = _MIN_DER + _softplus(d_lo_raw)
    d_hi = _MIN_DER + _softplus(d_hi_raw)
    d0 = _MIN_DER + _softplus(d0_raw)
    dK = _MIN_DER + _softplus(dK_raw)

    out_left = (ch0 + cy) - (e0 - z) * d0
    out_right = (z - eK) * dK + (chK + cy)
    lad_left = jnp.log(d0)
    lad_right = jnp.log(dK)

    theta = (zst - cw_lo) / w_b
    tmt = theta * (1.0 - theta)
    delta = h_b / w_b
    numer = h_b * (delta * theta * theta + d_lo * tmt)
    denom = delta + (d_lo + d_hi - 2.0 * delta) * tmt
    out_in = (ch_lo + cy) + numer / denom
    dnum = (delta * delta) * (d_hi * theta * theta + 2.0 * delta * tmt
                              + d_lo * (1.0 - theta) * (1.0 - theta))
    lad_in = jnp.log(dnum) - 2.0 * jnp.log(denom)

    out = jnp.where(lm, out_left, jnp.where(rm, out_right, out_in))
    lad = jnp.where(lm, lad_left, jnp.where(rm, lad_right, lad_in))
    out_ref[0, 0, :] = out
    lad_ref[0, 0, :] = lad


def kernel(inputs, unnormalized_widths, unnormalized_heights,
           unnormalized_derivatives, center_x, center_y):
    x = inputs.reshape(_G, 1, _R)
    cx = center_x.reshape(_G, 1, _R)
    cy = center_y.reshape(_G, 1, _R)
    uw = unnormalized_widths.reshape(_N, _K)
    uh = unnormalized_heights.reshape(_N, _K)
    ud = unnormalized_derivatives.reshape(_N, _K + 1)

    elem_spec = pl.BlockSpec((1, 1, _R), lambda i: (i, 0, 0))
    kspec = lambda k: pl.BlockSpec((_R, k), lambda i: (i, 0))

    out, lad = pl.pallas_call(
        _body,
        grid=(_G,),
        in_specs=[elem_spec, elem_spec, elem_spec,
                  kspec(_K), kspec(_K), kspec(_K + 1)],
        out_specs=[elem_spec, elem_spec],
        out_shape=[jax.ShapeDtypeStruct((_G, 1, _R), jnp.float32),
                   jax.ShapeDtypeStruct((_G, 1, _R), jnp.float32)],
    )(x, cx, cy, uw, uh, ud)
    return out.reshape(_B, _D), lad.reshape(_B, _D)


# speedup vs baseline: 4.2186x; 4.2186x over previous
import jax
import jax.numpy as jnp
from jax.experimental import pallas as pl

_B, _D, _K = 4096, 64, 32
_RB = 64
_G = _B // _RB
_R = _RB * _D
_MIN_BIN = 0.001
_MIN_DER = 0.001


def _softplus(v):
    return jnp.maximum(v, 0.0) + jnp.log1p(jnp.exp(-jnp.abs(v)))


def _t(a):
    return jax.lax.transpose(a, (1, 0))


def _body(x_ref, cx_ref, cy_ref, uw_ref, uh_ref, ud_ref, out_ref, lad_ref):
    K = _K
    x = x_ref[0, 0, :]
    cx = cx_ref[0, 0, :]
    cy = cy_ref[0, 0, :]
    z = x - cx

    spw = _MIN_BIN + _softplus(_t(uw_ref[...].reshape(_R, K)))
    sph = _MIN_BIN + _softplus(_t(uh_ref[...].reshape(_R, K)))
    udT = _t(ud_ref[...].reshape(_R, K + 1))

    jio = jax.lax.broadcasted_iota(jnp.int32, (K + 1, K), 0)
    kio = jax.lax.broadcasted_iota(jnp.int32, (K + 1, K), 1)
    MT = jnp.where(kio < jio, 0.5, -0.5).astype(jnp.float32)
    cw = jax.lax.dot_general(MT, spw, (((1,), (0,)), ((), ())),
                             preferred_element_type=jnp.float32,
                             precision=jax.lax.Precision.HIGHEST)
    ch = jax.lax.dot_general(MT, sph, (((1,), (0,)), ((), ())),
                             preferred_element_type=jnp.float32,
                             precision=jax.lax.Precision.HIGHEST)

    e0 = cw[0, :]
    eK = cw[K, :]
    ch0 = ch[0, :]
    chK = ch[K, :]

    lm = z < e0
    rm = z >= eK
    im = jnp.logical_not(jnp.logical_or(lm, rm))
    zst = jnp.where(im, z, 0.0)

    ind = (zst[None, :] >= cw).astype(jnp.float32)
    idx = jnp.sum(ind, axis=0).astype(jnp.int32) - 1

    sub33 = jax.lax.broadcasted_iota(jnp.int32, (K + 1, _R), 0)
    sub32 = jax.lax.broadcasted_iota(jnp.int32, (K, _R), 0)
    idx2 = idx[None, :]
    oh_lo33 = sub33 == idx2
    oh_hi33 = sub33 == idx2 + 1
    oh32 = sub32 == idx2

    def gat(mask, arr):
        return jnp.sum(jnp.where(mask, arr, 0.0), axis=0)

    cw_lo = gat(oh_lo33, cw)
    ch_lo = gat(oh_lo33, ch)
    w_b = gat(oh32, spw)
    h_b = gat(oh32, sph)
    d_lo_raw = gat(oh_lo33, udT)
    d_hi_raw = gat(oh_hi33, udT)
    d0_raw = udT[0, :]
    dK_raw = udT[K, :]

    d_lo = _MIN_DER + _softplus(d_lo_raw)
    d_hi = _MIN_DER + _softplus(d_hi_raw)
    d0 = _MIN_DER + _softplus(d0_raw)
    dK = _MIN_DER + _softplus(dK_raw)

    out_left = (ch0 + cy) - (e0 - z) * d0
    out_right = (z - eK) * dK + (chK + cy)
    lad_left = jnp.log(d0)
    lad_right = jnp.log(dK)

    theta = (zst - cw_lo) / w_b
    tmt = theta * (1.0 - theta)
    delta = h_b / w_b
    numer = h_b * (delta * theta * theta + d_lo * tmt)
    denom = delta + (d_lo + d_hi - 2.0 * delta) * tmt
    out_in = (ch_lo + cy) + numer / denom
    dnum = (delta * delta) * (d_hi * theta * theta + 2.0 * delta * tmt
                              + d_lo * (1.0 - theta) * (1.0 - theta))
    lad_in = jnp.log(dnum) - 2.0 * jnp.log(denom)

    out_ref[0, 0, :] = jnp.where(lm, out_left, jnp.where(rm, out_right, out_in))
    lad_ref[0, 0, :] = jnp.where(lm, lad_left, jnp.where(rm, lad_right, lad_in))


def kernel(inputs, unnormalized_widths, unnormalized_heights,
           unnormalized_derivatives, center_x, center_y):
    x = inputs.reshape(_G, 1, _R)
    cx = center_x.reshape(_G, 1, _R)
    cy = center_y.reshape(_G, 1, _R)

    espec = pl.BlockSpec((1, 1, _R), lambda i: (i, 0, 0))
    kspec = lambda k: pl.BlockSpec((_RB, _D, k), lambda i: (i, 0, 0))

    out, lad = pl.pallas_call(
        _body,
        grid=(_G,),
        in_specs=[espec, espec, espec,
                  kspec(_K), kspec(_K), kspec(_K + 1)],
        out_specs=[espec, espec],
        out_shape=[jax.ShapeDtypeStruct((_G, 1, _R), jnp.float32),
                   jax.ShapeDtypeStruct((_G, 1, _R), jnp.float32)],
    )(x, cx, cy,
      unnormalized_widths, unnormalized_heights, unnormalized_derivatives)
    return out.reshape(_B, _D), lad.reshape(_B, _D)


# speedup vs baseline: 4.3865x; 1.0398x over previous
import jax
import jax.numpy as jnp
from jax.experimental import pallas as pl

_B, _D, _K = 4096, 64, 32
_RB = 128
_G = _B // _RB
_R = _RB * _D
_MIN_BIN = 0.001
_MIN_DER = 0.001


def _softplus(v):
    return jnp.maximum(v, 0.0) + jnp.log1p(jnp.exp(-jnp.abs(v)))


def _t(a):
    return jax.lax.transpose(a, (1, 0))


def _body(x_ref, cx_ref, cy_ref, uw_ref, uh_ref, ud_ref, out_ref, lad_ref):
    K = _K
    x = x_ref[0, 0, :]
    cx = cx_ref[0, 0, :]
    cy = cy_ref[0, 0, :]
    z = x - cx

    spw = _MIN_BIN + _softplus(_t(uw_ref[...].reshape(_R, K)))
    sph = _MIN_BIN + _softplus(_t(uh_ref[...].reshape(_R, K)))
    udT = _t(ud_ref[...].reshape(_R, K + 1))

    jio = jax.lax.broadcasted_iota(jnp.int32, (K + 1, K), 0)
    kio = jax.lax.broadcasted_iota(jnp.int32, (K + 1, K), 1)
    MT = jnp.where(kio < jio, 0.5, -0.5).astype(jnp.float32)
    cw = jax.lax.dot_general(MT, spw, (((1,), (0,)), ((), ())),
                             preferred_element_type=jnp.float32,
                             precision=jax.lax.Precision.HIGHEST)
    ch = jax.lax.dot_general(MT, sph, (((1,), (0,)), ((), ())),
                             preferred_element_type=jnp.float32,
                             precision=jax.lax.Precision.HIGHEST)

    e0 = cw[0, :]
    eK = cw[K, :]
    ch0 = ch[0, :]
    chK = ch[K, :]

    lm = z < e0
    rm = z >= eK
    im = jnp.logical_not(jnp.logical_or(lm, rm))
    zst = jnp.where(im, z, 0.0)

    ind = (zst[None, :] >= cw).astype(jnp.float32)
    idx = jnp.sum(ind, axis=0).astype(jnp.int32) - 1

    sub33 = jax.lax.broadcasted_iota(jnp.int32, (K + 1, _R), 0)
    sub32 = jax.lax.broadcasted_iota(jnp.int32, (K, _R), 0)
    idx2 = idx[None, :]
    oh_lo33 = sub33 == idx2
    oh_hi33 = sub33 == idx2 + 1
    oh32 = sub32 == idx2

    def gat(mask, arr):
        return jnp.sum(jnp.where(mask, arr, 0.0), axis=0)

    cw_lo = gat(oh_lo33, cw)
    ch_lo = gat(oh_lo33, ch)
    w_b = gat(oh32, spw)
    h_b = gat(oh32, sph)
    d_lo_raw = gat(oh_lo33, udT)
    d_hi_raw = gat(oh_hi33, udT)
    d0_raw = udT[0, :]
    dK_raw = udT[K, :]

    d_lo = _MIN_DER + _softplus(d_lo_raw)
    d_hi = _MIN_DER + _softplus(d_hi_raw)
    d0 = _MIN_DER + _softplus(d0_raw)
    dK = _MIN_DER + _softplus(dK_raw)

    out_left = (ch0 + cy) - (e0 - z) * d0
    out_right = (z - eK) * dK + (chK + cy)
    lad_left = jnp.log(d0)
    lad_right = jnp.log(dK)

    theta = (zst - cw_lo) / w_b
    tmt = theta * (1.0 - theta)
    delta = h_b / w_b
    numer = h_b * (delta * theta * theta + d_lo * tmt)
    denom = delta + (d_lo + d_hi - 2.0 * delta) * tmt
    out_in = (ch_lo + cy) + numer / denom
    dnum = (delta * delta) * (d_hi * theta * theta + 2.0 * delta * tmt
                              + d_lo * (1.0 - theta) * (1.0 - theta))
    lad_in = jnp.log(dnum) - 2.0 * jnp.log(denom)

    out_ref[0, 0, :] = jnp.where(lm, out_left, jnp.where(rm, out_right, out_in))
    lad_ref[0, 0, :] = jnp.where(lm, lad_left, jnp.where(rm, lad_right, lad_in))


def kernel(inputs, unnormalized_widths, unnormalized_heights,
           unnormalized_derivatives, center_x, center_y):
    x = inputs.reshape(_G, 1, _R)
    cx = center_x.reshape(_G, 1, _R)
    cy = center_y.reshape(_G, 1, _R)

    espec = pl.BlockSpec((1, 1, _R), lambda i: (i, 0, 0))
    kspec = lambda k: pl.BlockSpec((_RB, _D, k), lambda i: (i, 0, 0))

    out, lad = pl.pallas_call(
        _body,
        grid=(_G,),
        in_specs=[espec, espec, espec,
                  kspec(_K), kspec(_K), kspec(_K + 1)],
        out_specs=[espec, espec],
        out_shape=[jax.ShapeDtypeStruct((_G, 1, _R), jnp.float32),
                   jax.ShapeDtypeStruct((_G, 1, _R), jnp.float32)],
    )(x, cx, cy,
      unnormalized_widths, unnormalized_heights, unnormalized_derivatives)
    return out.reshape(_B, _D), lad.reshape(_B, _D)


# speedup vs baseline: 5.9860x; 1.3646x over previous
import jax
import jax.numpy as jnp
from jax.experimental import pallas as pl

_B, _D, _K = 4096, 64, 32
_RB = 128
_G = _B // _RB
_MIN_BIN = 0.001
_MIN_DER = 0.001


def _softplus(v):
    return jnp.maximum(v, 0.0) + jnp.log1p(jnp.exp(-jnp.abs(v)))


def _t(a):
    return jax.lax.transpose(a, (1, 0))


def _body(x_ref, cx_ref, cy_ref, uw_ref, uh_ref, ud_ref, out_ref, lad_ref):
    K = _K
    x = _t(x_ref[...])
    cx = _t(cx_ref[...])
    cy = _t(cy_ref[...])
    z = x - cx

    spw = _MIN_BIN + _softplus(_t(uw_ref[...]).reshape(_D, K, _RB))
    sph = _MIN_BIN + _softplus(_t(uh_ref[...]).reshape(_D, K, _RB))
    ud3 = _t(ud_ref[...]).reshape(_D, K + 1, _RB)

    def csum(c):
        for s in (1, 2, 4, 8, 16):
            sh = jnp.concatenate(
                [jnp.zeros((_D, s, _RB), jnp.float32), c[:, :-s, :]], axis=1)
            c = c + sh
        return c

    cw = csum(spw)
    chh = csum(sph)
    totw = cw[:, K - 1:K, :]
    toth = chh[:, K - 1:K, :]
    ecw = jnp.concatenate([jnp.zeros((_D, 1, _RB), jnp.float32), cw],
                          axis=1) - 0.5 * totw
    ech = jnp.concatenate([jnp.zeros((_D, 1, _RB), jnp.float32), chh],
                          axis=1) - 0.5 * toth

    e0 = ecw[:, 0, :]
    eK = ecw[:, K, :]
    ch0 = ech[:, 0, :]
    chK = ech[:, K, :]

    lm = z < e0
    rm = z >= eK
    im = jnp.logical_not(jnp.logical_or(lm, rm))
    zst = jnp.where(im, z, 0.0)
    zst3 = zst[:, None, :]

    ind = (zst3 >= ecw).astype(jnp.float32)
    idx = jnp.sum(ind, axis=1).astype(jnp.int32) - 1
    idx3 = idx[:, None, :]

    k33 = jax.lax.broadcasted_iota(jnp.int32, (_D, K + 1, _RB), 1)
    k32 = jax.lax.broadcasted_iota(jnp.int32, (_D, K, _RB), 1)
    oh_lo33 = k33 == idx3
    oh_hi33 = k33 == idx3 + 1
    oh32 = k32 == idx3

    def gat(mask, arr):
        return jnp.sum(jnp.where(mask, arr, 0.0), axis=1)

    cw_lo = gat(oh_lo33, ecw)
    ch_lo = gat(oh_lo33, ech)
    w_b = gat(oh32, spw)
    h_b = gat(oh32, sph)
    d_lo_raw = gat(oh_lo33, ud3)
    d_hi_raw = gat(oh_hi33, ud3)
    d0_raw = ud3[:, 0, :]
    dK_raw = ud3[:, K, :]

    d_lo = _MIN_DER + _softplus(d_lo_raw)
    d_hi = _MIN_DER + _softplus(d_hi_raw)
    d0 = _MIN_DER + _softplus(d0_raw)
    dK = _MIN_DER + _softplus(dK_raw)

    out_left = (ch0 + cy) - (e0 - z) * d0
    out_right = (z - eK) * dK + (chK + cy)
    lad_left = jnp.log(d0)
    lad_right = jnp.log(dK)

    theta = (zst - cw_lo) / w_b
    tmt = theta * (1.0 - theta)
    delta = h_b / w_b
    numer = h_b * (delta * theta * theta + d_lo * tmt)
    denom = delta + (d_lo + d_hi - 2.0 * delta) * tmt
    out_in = (ch_lo + cy) + numer / denom
    dnum = (delta * delta) * (d_hi * theta * theta + 2.0 * delta * tmt
                              + d_lo * (1.0 - theta) * (1.0 - theta))
    lad_in = jnp.log(dnum) - 2.0 * jnp.log(denom)

    out = jnp.where(lm, out_left, jnp.where(rm, out_right, out_in))
    lad = jnp.where(lm, lad_left, jnp.where(rm, lad_right, lad_in))
    out_ref[...] = _t(out)
    lad_ref[...] = _t(lad)


def kernel(inputs, unnormalized_widths, unnormalized_heights,
           unnormalized_derivatives, center_x, center_y):
    uw = unnormalized_widths.reshape(_B, _D * _K)
    uh = unnormalized_heights.reshape(_B, _D * _K)
    ud = unnormalized_derivatives.reshape(_B, _D * (_K + 1))

    espec = pl.BlockSpec((_RB, _D), lambda i: (i, 0))
    kspec = lambda k: pl.BlockSpec((_RB, _D * k), lambda i: (i, 0))

    out, lad = pl.pallas_call(
        _body,
        grid=(_G,),
        in_specs=[espec, espec, espec,
                  kspec(_K), kspec(_K), kspec(_K + 1)],
        out_specs=[espec, espec],
        out_shape=[jax.ShapeDtypeStruct((_B, _D), jnp.float32),
                   jax.ShapeDtypeStruct((_B, _D), jnp.float32)],
    )(inputs, center_x, center_y, uw, uh, ud)
    return out, lad


# speedup vs baseline: 6.2390x; 1.0423x over previous
import jax
import jax.numpy as jnp
from jax.experimental import pallas as pl

_B, _D, _K = 4096, 64, 32
_RB = 256
_G = _B // _RB
_MIN_BIN = 0.001
_MIN_DER = 0.001


def _softplus(v):
    return jnp.maximum(v, 0.0) + jnp.log1p(jnp.exp(-jnp.abs(v)))


def _t(a):
    return jax.lax.transpose(a, (1, 0))


def _body(x_ref, cx_ref, cy_ref, uw_ref, uh_ref, ud_ref, out_ref, lad_ref):
    K = _K
    x = _t(x_ref[...])
    cx = _t(cx_ref[...])
    cy = _t(cy_ref[...])
    z = x - cx

    spw = _MIN_BIN + _softplus(_t(uw_ref[...]).reshape(_D, K, _RB))
    sph = _MIN_BIN + _softplus(_t(uh_ref[...]).reshape(_D, K, _RB))
    ud3 = _t(ud_ref[...]).reshape(_D, K + 1, _RB)

    def csum(c):
        for s in (1, 2, 4, 8, 16):
            sh = jnp.concatenate(
                [jnp.zeros((_D, s, _RB), jnp.float32), c[:, :-s, :]], axis=1)
            c = c + sh
        return c

    cw = csum(spw)
    chh = csum(sph)
    totw = cw[:, K - 1:K, :]
    toth = chh[:, K - 1:K, :]
    ecw = jnp.concatenate([jnp.zeros((_D, 1, _RB), jnp.float32), cw],
                          axis=1) - 0.5 * totw
    ech = jnp.concatenate([jnp.zeros((_D, 1, _RB), jnp.float32), chh],
                          axis=1) - 0.5 * toth

    e0 = ecw[:, 0, :]
    eK = ecw[:, K, :]
    ch0 = ech[:, 0, :]
    chK = ech[:, K, :]

    lm = z < e0
    rm = z >= eK
    im = jnp.logical_not(jnp.logical_or(lm, rm))
    zst = jnp.where(im, z, 0.0)
    zst3 = zst[:, None, :]

    ind = (zst3 >= ecw).astype(jnp.float32)
    idx = jnp.sum(ind, axis=1).astype(jnp.int32) - 1
    idx3 = idx[:, None, :]

    k33 = jax.lax.broadcasted_iota(jnp.int32, (_D, K + 1, _RB), 1)
    k32 = jax.lax.broadcasted_iota(jnp.int32, (_D, K, _RB), 1)
    oh_lo33 = k33 == idx3
    oh_hi33 = k33 == idx3 + 1
    oh32 = k32 == idx3

    def gat(mask, arr):
        return jnp.sum(jnp.where(mask, arr, 0.0), axis=1)

    cw_lo = gat(oh_lo33, ecw)
    ch_lo = gat(oh_lo33, ech)
    w_b = gat(oh32, spw)
    h_b = gat(oh32, sph)
    d_lo_raw = gat(oh_lo33, ud3)
    d_hi_raw = gat(oh_hi33, ud3)
    d0_raw = ud3[:, 0, :]
    dK_raw = ud3[:, K, :]

    d_lo = _MIN_DER + _softplus(d_lo_raw)
    d_hi = _MIN_DER + _softplus(d_hi_raw)
    d0 = _MIN_DER + _softplus(d0_raw)
    dK = _MIN_DER + _softplus(dK_raw)

    out_left = (ch0 + cy) - (e0 - z) * d0
    out_right = (z - eK) * dK + (chK + cy)
    lad_left = jnp.log(d0)
    lad_right = jnp.log(dK)

    theta = (zst - cw_lo) / w_b
    tmt = theta * (1.0 - theta)
    delta = h_b / w_b
    numer = h_b * (delta * theta * theta + d_lo * tmt)
    denom = delta + (d_lo + d_hi - 2.0 * delta) * tmt
    out_in = (ch_lo + cy) + numer / denom
    dnum = (delta * delta) * (d_hi * theta * theta + 2.0 * delta * tmt
                              + d_lo * (1.0 - theta) * (1.0 - theta))
    lad_in = jnp.log(dnum) - 2.0 * jnp.log(denom)

    out = jnp.where(lm, out_left, jnp.where(rm, out_right, out_in))
    lad = jnp.where(lm, lad_left, jnp.where(rm, lad_right, lad_in))
    out_ref[...] = _t(out)
    lad_ref[...] = _t(lad)


def kernel(inputs, unnormalized_widths, unnormalized_heights,
           unnormalized_derivatives, center_x, center_y):
    uw = unnormalized_widths.reshape(_B, _D * _K)
    uh = unnormalized_heights.reshape(_B, _D * _K)
    ud = unnormalized_derivatives.reshape(_B, _D * (_K + 1))

    espec = pl.BlockSpec((_RB, _D), lambda i: (i, 0))
    kspec = lambda k: pl.BlockSpec((_RB, _D * k), lambda i: (i, 0))

    out, lad = pl.pallas_call(
        _body,
        grid=(_G,),
        in_specs=[espec, espec, espec,
                  kspec(_K), kspec(_K), kspec(_K + 1)],
        out_specs=[espec, espec],
        out_shape=[jax.ShapeDtypeStruct((_B, _D), jnp.float32),
                   jax.ShapeDtypeStruct((_B, _D), jnp.float32)],
    )(inputs, center_x, center_y, uw, uh, ud)
    return out, lad
